# baseline (speedup 1.0000x reference)
import jax
import jax.numpy as jnp
from jax import lax
from jax.experimental import pallas as pl
from jax.experimental.pallas import tpu as pltpu
from jax.experimental.pallas import tpu_sc as plsc

N = 320000
D = 128
NV = D // 16
B = 10000
NW = 32
SEG_BASE = 312
SEG_MAX = 328
RB = 256
CH = 16
NEGF = -3.0e38


def _sc_body(x_hbm, ids_hbm, bounds_hbm, out_hbm,
             bvec, xbuf, idbuf, sumacc, maxacc, cntbuf, stage):
    wid = lax.axis_index("s") * 2 + lax.axis_index("c")
    seg_lo = wid * SEG_BASE
    is_last = (wid == NW - 1).astype(jnp.int32)
    nchunks = SEG_BASE // 8 + is_last * ((SEG_MAX - SEG_BASE) // 8)

    pltpu.sync_copy(bounds_hbm.at[wid], bvec)
    bv = bvec[...]
    lane = lax.broadcasted_iota(jnp.int32, (16,), 0)
    one_hot0 = 1 - jnp.minimum(lane, 1)
    rs = bv[0]
    re = bv[1]

    zero16 = jnp.zeros((16,), jnp.float32)
    ninf16 = jnp.full((16,), NEGF, jnp.float32)
    zcnt = jnp.zeros((16,), jnp.int32)

    def init_body(i, _):
        for j in range(NV):
            sumacc[i, pl.ds(16 * j, 16)] = zero16
            maxacc[i, pl.ds(16 * j, 16)] = ninf16
        return 0
    lax.fori_loop(0, SEG_MAX, init_body, 0)

    def cinit_body(i, _):
        cntbuf[pl.ds(16 * i, 16)] = zcnt
        return 0
    lax.fori_loop(0, (SEG_MAX + 16) // 16, cinit_body, 0)

    def flush_maxcnt(lid_c, cnt_c, maxs):
        for j in range(NV):
            sl = pl.ds(16 * j, 16)
            m_old = maxacc[lid_c, sl]
            maxacc[lid_c, sl] = jnp.maximum(m_old, maxs[j])
        plsc.addupdate(cntbuf.at[pl.ds(lid_c, 16)], one_hot0 * cnt_c)

    def chunk_uniform(cb):
        idv = idbuf[pl.ds(cb, 16)]
        lid = idv[0] - seg_lo
        sums = [zero16] * NV
        maxs = [ninf16] * NV
        for rr in range(CH):
            xs = [xbuf[cb + rr, pl.ds(16 * j, 16)] for j in range(NV)]
            sums = [sums[j] + xs[j] for j in range(NV)]
            maxs = [jnp.maximum(maxs[j], xs[j]) for j in range(NV)]
        for j in range(NV):
            sl = pl.ds(16 * j, 16)
            plsc.addupdate(sumacc.at[lid, sl], sums[j])
            m_old = maxacc[lid, sl]
            maxacc[lid, sl] = jnp.maximum(m_old, maxs[j])
        plsc.addupdate(cntbuf.at[pl.ds(lid, 16)], one_hot0 * CH)

    def chunk_work(cb, lo_r, hi_r, masked):
        idv = idbuf[pl.ds(cb, 16)]
        lid_c = jnp.int32(-1)
        cnt_c = jnp.int32(0)
        maxs = [ninf16] * NV
        for rr in range(CH):
            r = cb + rr
            nlid = idv[rr] - seg_lo
            xs = [xbuf[r, pl.ds(16 * j, 16)] for j in range(NV)]
            if masked:
                val_i = ((r >= lo_r) & (r < hi_r)).astype(jnp.int32)
                ch = (nlid != lid_c).astype(jnp.int32) * val_i
            else:
                val_i = jnp.int32(1)
                ch = (nlid != lid_c).astype(jnp.int32)
            ch_b = ch > 0

            @pl.when(jnp.logical_and(ch_b, lid_c >= 0))
            def _(lid_c=lid_c, cnt_c=cnt_c, maxs=maxs):
                flush_maxcnt(lid_c, cnt_c, maxs)

            ch_f = ch.astype(jnp.float32)
            alive = jnp.full((16,), 1.0 - ch_f, jnp.float32)
            pen = jnp.full((16,), ch_f * NEGF, jnp.float32)
            if masked:
                val_f = val_i.astype(jnp.float32)
                vgate = jnp.full((16,), val_f, jnp.float32)
                vpen = jnp.full((16,), (1.0 - val_f) * NEGF, jnp.float32)
                xg = [xs[j] * vgate for j in range(NV)]
                maxs = [jnp.maximum(maxs[j] * alive + pen, xg[j] + vpen)
                        for j in range(NV)]
                lid_s = nlid * val_i
                lid_c = nlid * val_i + lid_c * (1 - val_i)
            else:
                xg = xs
                maxs = [jnp.maximum(maxs[j] * alive + pen, xs[j])
                        for j in range(NV)]
                lid_s = nlid
                lid_c = nlid
            for j in range(NV):
                plsc.addupdate(sumacc.at[lid_s, pl.ds(16 * j, 16)], xg[j])
            cnt_c = cnt_c * (1 - ch) + val_i

        @pl.when(lid_c >= 0)
        def _():
            flush_maxcnt(lid_c, cnt_c, maxs)

    def block_body(kb, _):
        base = kb * RB
        pltpu.sync_copy(x_hbm.at[pl.ds(base, RB)], xbuf)
        pltpu.sync_copy(ids_hbm.at[pl.ds(base, RB)], idbuf.at[pl.ds(0, RB)])
        lo_r = jnp.maximum(rs - base, 0)
        hi_r = jnp.minimum(re - base, RB)

        def chunk_body(c, _):
            cb = c * CH
            full = jnp.logical_and(cb >= lo_r, cb + CH <= hi_r)
            idv0 = idbuf[pl.ds(cb, 16)]
            uniform = jnp.logical_and(full, idv0[0] == idv0[CH - 1])

            @pl.when(uniform)
            def _():
                chunk_uniform(cb)

            @pl.when(jnp.logical_and(full, jnp.logical_not(uniform)))
            def _():
                chunk_work(cb, lo_r, hi_r, False)

            @pl.when(jnp.logical_not(full))
            def _():
                chunk_work(cb, lo_r, hi_r, True)
            return 0

        lax.fori_loop(lo_r // CH, (hi_r + CH - 1) // CH, chunk_body, 0)
        return 0

    kb_lo = rs // RB
    kb_hi = (re + RB - 1) // RB
    lax.fori_loop(kb_lo, kb_hi, block_body, 0)

    def fin_body(cidx, _):
        cload = cntbuf[pl.ds(cidx * 8, 16)]
        for s in range(8):
            row = cidx * 8 + s
            cs = cload[s]
            denom = jnp.maximum(jnp.full((16,), cs, jnp.int32),
                                1).astype(jnp.float32)
            for j in range(NV):
                sl = pl.ds(16 * j, 16)
                sv = sumacc[row, sl]
                stage[s, sl] = sv
                stage[s, pl.ds(D + 16 * j, 16)] = sv / denom
                stage[s, pl.ds(2 * D + 16 * j, 16)] = maxacc[row, sl]

            @pl.when(cs == 0)
            def _zero_row():
                for j in range(3 * NV):
                    stage[s, pl.ds(16 * j, 16)] = zero16
        pltpu.sync_copy(stage, out_hbm.at[pl.ds(seg_lo + cidx * 8, 8)])
        return 0
    lax.fori_loop(0, nchunks, fin_body, 0)


@jax.jit
def _run(x, batch_i32, bounds):
    mesh = plsc.VectorSubcoreMesh(core_axis_name="c", subcore_axis_name="s")
    f = pl.kernel(
        _sc_body,
        out_type=jax.ShapeDtypeStruct((B, 3 * D), jnp.float32),
        mesh=mesh,
        scratch_types=[
            pltpu.VMEM((16,), jnp.int32),
            pltpu.VMEM((RB, D), jnp.float32),
            pltpu.VMEM((RB + 16,), jnp.int32),
            pltpu.VMEM((SEG_MAX, D), jnp.float32),
            pltpu.VMEM((SEG_MAX, D), jnp.float32),
            pltpu.VMEM((SEG_MAX + 16,), jnp.int32),
            pltpu.VMEM((8, 3 * D), jnp.float32),
        ],
    )
    return f(x, batch_i32, bounds)


def kernel(x, batch, batch_size):
    ids = batch.astype(jnp.int32)
    bvals = jnp.array([SEG_BASE * w for w in range(NW)] + [B], jnp.int32)
    bnds = jnp.searchsorted(ids, bvals, side="left").astype(jnp.int32)
    bounds = jnp.zeros((NW, 16), jnp.int32)
    bounds = bounds.at[:, 0].set(bnds[:NW]).at[:, 1].set(bnds[1:])
    return _run(x, ids, bounds)


# speedup vs baseline: 1.7462x; 1.7462x over previous
import jax
import jax.numpy as jnp
from jax import lax
from jax.experimental import pallas as pl
from jax.experimental.pallas import tpu as pltpu
from jax.experimental.pallas import tpu_sc as plsc

N = 320000
D = 128
NV = D // 16
B = 10000
NW = 32
SEG_BASE = 312
SEG_MAX = 328
RB = 256
CH = 16
NEGF = -3.0e38


def _sc_body(x_hbm, ids_hbm, bounds_hbm, out_hbm,
             bvec, xbuf, idbuf, sumacc, maxacc, cntbuf, stage):
    wid = lax.axis_index("s") * 2 + lax.axis_index("c")
    seg_lo = wid * SEG_BASE
    is_last = (wid == NW - 1).astype(jnp.int32)
    nchunks = SEG_BASE // 8 + is_last * ((SEG_MAX - SEG_BASE) // 8)

    pltpu.sync_copy(bounds_hbm.at[wid], bvec)
    bv = bvec[...]
    lane = lax.broadcasted_iota(jnp.int32, (16,), 0)
    one_hot0 = 1 - jnp.minimum(lane, 1)
    rs = bv[0]
    re = bv[1]

    zero16 = jnp.zeros((16,), jnp.float32)
    ninf16 = jnp.full((16,), NEGF, jnp.float32)
    zcnt = jnp.zeros((16,), jnp.int32)

    def init_body(i, _):
        for j in range(NV):
            sumacc[i, pl.ds(16 * j, 16)] = zero16
            maxacc[i, pl.ds(16 * j, 16)] = ninf16
        return 0
    lax.fori_loop(0, SEG_MAX, init_body, 0)

    def cinit_body(i, _):
        cntbuf[pl.ds(16 * i, 16)] = zcnt
        return 0
    lax.fori_loop(0, (SEG_MAX + 16) // 16, cinit_body, 0)

    def flush_maxcnt(lid_c, cnt_c, maxs):
        for j in range(NV):
            sl = pl.ds(16 * j, 16)
            m_old = maxacc[lid_c, sl]
            maxacc[lid_c, sl] = jnp.maximum(m_old, maxs[j])
        plsc.addupdate(cntbuf.at[pl.ds(lid_c, 16)], one_hot0 * cnt_c)

    def chunk_uniform(cb):
        idv = idbuf[pl.ds(cb, 16)]
        lid = idv[0] - seg_lo
        sums = [zero16] * NV
        maxs = [ninf16] * NV
        for rr in range(CH):
            xs = [xbuf[cb + rr, pl.ds(16 * j, 16)] for j in range(NV)]
            sums = [sums[j] + xs[j] for j in range(NV)]
            maxs = [jnp.maximum(maxs[j], xs[j]) for j in range(NV)]
        for j in range(NV):
            sl = pl.ds(16 * j, 16)
            plsc.addupdate(sumacc.at[lid, sl], sums[j])
            m_old = maxacc[lid, sl]
            maxacc[lid, sl] = jnp.maximum(m_old, maxs[j])
        plsc.addupdate(cntbuf.at[pl.ds(lid, 16)], one_hot0 * CH)

    def flush(lid_c, cnt_c, sums, maxs):
        for j in range(NV):
            sl = pl.ds(16 * j, 16)
            plsc.addupdate(sumacc.at[lid_c, sl], sums[j])
            m_old = maxacc[lid_c, sl]
            maxacc[lid_c, sl] = jnp.maximum(m_old, maxs[j])
        plsc.addupdate(cntbuf.at[pl.ds(lid_c, 16)], one_hot0 * cnt_c)

    def chunk_work(cb, lo_r, hi_r, masked):
        idv = idbuf[pl.ds(cb, 16)]
        lid_c = jnp.int32(-1)
        cnt_c = jnp.int32(0)
        sums = [zero16] * NV
        maxs = [ninf16] * NV
        for rr in range(CH):
            r = cb + rr
            nlid = idv[rr] - seg_lo
            xs = [xbuf[r, pl.ds(16 * j, 16)] for j in range(NV)]
            if masked:
                val_i = ((r >= lo_r) & (r < hi_r)).astype(jnp.int32)
                ch = (nlid != lid_c).astype(jnp.int32) * val_i
            else:
                val_i = jnp.int32(1)
                ch = (nlid != lid_c).astype(jnp.int32)
            ch_b = ch > 0

            @pl.when(jnp.logical_and(ch_b, lid_c >= 0))
            def _(lid_c=lid_c, cnt_c=cnt_c, sums=sums, maxs=maxs):
                flush(lid_c, cnt_c, sums, maxs)

            ch_f = ch.astype(jnp.float32)
            alive = jnp.full((16,), 1.0 - ch_f, jnp.float32)
            pen = jnp.full((16,), ch_f * NEGF, jnp.float32)
            if masked:
                val_f = val_i.astype(jnp.float32)
                vgate = jnp.full((16,), val_f, jnp.float32)
                vpen = jnp.full((16,), (1.0 - val_f) * NEGF, jnp.float32)
                sums = [sums[j] * alive + xs[j] * vgate for j in range(NV)]
                maxs = [jnp.maximum(maxs[j] * alive + pen,
                                    xs[j] * vgate + vpen) for j in range(NV)]
                lid_c = nlid * val_i + lid_c * (1 - val_i)
            else:
                sums = [sums[j] * alive + xs[j] for j in range(NV)]
                maxs = [jnp.maximum(maxs[j] * alive + pen, xs[j])
                        for j in range(NV)]
                lid_c = nlid
            cnt_c = cnt_c * (1 - ch) + val_i

        @pl.when(lid_c >= 0)
        def _():
            flush(lid_c, cnt_c, sums, maxs)

    def block_body(kb, _):
        base = kb * RB
        pltpu.sync_copy(x_hbm.at[pl.ds(base, RB)], xbuf)
        pltpu.sync_copy(ids_hbm.at[pl.ds(base, RB)], idbuf.at[pl.ds(0, RB)])
        lo_r = jnp.maximum(rs - base, 0)
        hi_r = jnp.minimum(re - base, RB)

        def chunk_body(c, _):
            cb = c * CH
            full = jnp.logical_and(cb >= lo_r, cb + CH <= hi_r)
            idv0 = idbuf[pl.ds(cb, 16)]
            uniform = jnp.logical_and(full, idv0[0] == idv0[CH - 1])

            @pl.when(uniform)
            def _():
                chunk_uniform(cb)

            @pl.when(jnp.logical_and(full, jnp.logical_not(uniform)))
            def _():
                chunk_work(cb, lo_r, hi_r, False)

            @pl.when(jnp.logical_not(full))
            def _():
                chunk_work(cb, lo_r, hi_r, True)
            return 0

        lax.fori_loop(lo_r // CH, (hi_r + CH - 1) // CH, chunk_body, 0)
        return 0

    kb_lo = rs // RB
    kb_hi = (re + RB - 1) // RB
    lax.fori_loop(kb_lo, kb_hi, block_body, 0)

    def fin_body(cidx, _):
        cload = cntbuf[pl.ds(cidx * 8, 16)]
        for s in range(8):
            row = cidx * 8 + s
            cs = cload[s]
            denom = jnp.maximum(jnp.full((16,), cs, jnp.int32),
                                1).astype(jnp.float32)
            for j in range(NV):
                sl = pl.ds(16 * j, 16)
                sv = sumacc[row, sl]
                stage[s, sl] = sv
                stage[s, pl.ds(D + 16 * j, 16)] = sv / denom
                stage[s, pl.ds(2 * D + 16 * j, 16)] = maxacc[row, sl]

            @pl.when(cs == 0)
            def _zero_row():
                for j in range(3 * NV):
                    stage[s, pl.ds(16 * j, 16)] = zero16
        pltpu.sync_copy(stage, out_hbm.at[pl.ds(seg_lo + cidx * 8, 8)])
        return 0
    lax.fori_loop(0, nchunks, fin_body, 0)


@jax.jit
def _run(x, batch_i32, bounds):
    mesh = plsc.VectorSubcoreMesh(core_axis_name="c", subcore_axis_name="s")
    f = pl.kernel(
        _sc_body,
        out_type=jax.ShapeDtypeStruct((B, 3 * D), jnp.float32),
        mesh=mesh,
        scratch_types=[
            pltpu.VMEM((16,), jnp.int32),
            pltpu.VMEM((RB, D), jnp.float32),
            pltpu.VMEM((RB + 16,), jnp.int32),
            pltpu.VMEM((SEG_MAX, D), jnp.float32),
            pltpu.VMEM((SEG_MAX, D), jnp.float32),
            pltpu.VMEM((SEG_MAX + 16,), jnp.int32),
            pltpu.VMEM((8, 3 * D), jnp.float32),
        ],
    )
    return f(x, batch_i32, bounds)


def kernel(x, batch, batch_size):
    ids = batch.astype(jnp.int32)
    bvals = jnp.array([SEG_BASE * w for w in range(NW)] + [B], jnp.int32)
    bnds = jnp.searchsorted(ids, bvals, side="left").astype(jnp.int32)
    bounds = jnp.zeros((NW, 16), jnp.int32)
    bounds = bounds.at[:, 0].set(bnds[:NW]).at[:, 1].set(bnds[1:])
    return _run(x, ids, bounds)


# speedup vs baseline: 3.3258x; 1.9046x over previous
import jax
import jax.numpy as jnp
from jax import lax
from jax.experimental import pallas as pl
from jax.experimental.pallas import tpu as pltpu
from jax.experimental.pallas import tpu_sc as plsc

N = 320000
D = 128
NV = D // 16
B = 10000
NW = 32
SEG_BASE = 312
SEG_MAX = 328
RB = 128
CH = 16
NEGF = -3.0e38


def _sc_body(x_hbm, ids_hbm, bounds_hbm, out_hbm,
             bvec, xbuf0, xbuf1, idbuf0, idbuf1,
             sumacc, maxacc, cntbuf, stage, sem0, sem1):
    wid = lax.axis_index("s") * 2 + lax.axis_index("c")
    seg_lo = wid * SEG_BASE
    is_last = (wid == NW - 1).astype(jnp.int32)
    nchunks = SEG_BASE // 8 + is_last * ((SEG_MAX - SEG_BASE) // 8)

    pltpu.sync_copy(bounds_hbm.at[wid], bvec)
    bv = bvec[...]
    lane = lax.broadcasted_iota(jnp.int32, (16,), 0)
    one_hot0 = 1 - jnp.minimum(lane, 1)
    rs = bv[0]
    re = bv[1]

    zero16 = jnp.zeros((16,), jnp.float32)
    ninf16 = jnp.full((16,), NEGF, jnp.float32)
    zcnt = jnp.zeros((16,), jnp.int32)

    def init_body(i, _):
        for j in range(NV):
            sumacc[i, pl.ds(16 * j, 16)] = zero16
            maxacc[i, pl.ds(16 * j, 16)] = ninf16
        return 0
    lax.fori_loop(0, SEG_MAX, init_body, 0)

    def cinit_body(i, _):
        cntbuf[pl.ds(16 * i, 16)] = zcnt
        return 0
    lax.fori_loop(0, (SEG_MAX + 16) // 16, cinit_body, 0)

    def flush(lid_c, cnt_c, sums, maxs):
        for j in range(NV):
            sl = pl.ds(16 * j, 16)
            plsc.addupdate(sumacc.at[lid_c, sl], sums[j])
            m_old = maxacc[lid_c, sl]
            maxacc[lid_c, sl] = jnp.maximum(m_old, maxs[j])
        plsc.addupdate(cntbuf.at[pl.ds(lid_c, 16)], one_hot0 * cnt_c)

    def chunk_work(xb, ib, cb, lo_r, hi_r, masked):
        idv = ib[pl.ds(cb, 16)]
        lid_c = jnp.int32(-1)
        cnt_c = jnp.int32(0)
        sums = [zero16] * NV
        maxs = [ninf16] * NV
        for rr in range(CH):
            r = cb + rr
            nlid = idv[rr] - seg_lo
            xs = [xb[r, pl.ds(16 * j, 16)] for j in range(NV)]
            if masked:
                val_i = ((r >= lo_r) & (r < hi_r)).astype(jnp.int32)
                ch = (nlid != lid_c).astype(jnp.int32) * val_i
            else:
                val_i = jnp.int32(1)
                ch = (nlid != lid_c).astype(jnp.int32)
            ch_b = ch > 0

            @pl.when(jnp.logical_and(ch_b, lid_c >= 0))
            def _(lid_c=lid_c, cnt_c=cnt_c, sums=sums, maxs=maxs):
                flush(lid_c, cnt_c, sums, maxs)

            ch_f = ch.astype(jnp.float32)
            alive = jnp.full((16,), 1.0 - ch_f, jnp.float32)
            pen = jnp.full((16,), ch_f * NEGF, jnp.float32)
            if masked:
                val_f = val_i.astype(jnp.float32)
                vgate = jnp.full((16,), val_f, jnp.float32)
                vpen = jnp.full((16,), (1.0 - val_f) * NEGF, jnp.float32)
                sums = [sums[j] * alive + xs[j] * vgate for j in range(NV)]
                maxs = [jnp.maximum(maxs[j] * alive + pen,
                                    xs[j] * vgate + vpen) for j in range(NV)]
                lid_c = nlid * val_i + lid_c * (1 - val_i)
            else:
                sums = [sums[j] * alive + xs[j] for j in range(NV)]
                maxs = [jnp.maximum(maxs[j] * alive + pen, xs[j])
                        for j in range(NV)]
                lid_c = nlid
            cnt_c = cnt_c * (1 - ch) + val_i

        @pl.when(lid_c >= 0)
        def _():
            flush(lid_c, cnt_c, sums, maxs)

    def start_dma(kb, xb, ib, sem):
        base = kb * RB
        pltpu.async_copy(x_hbm.at[pl.ds(base, RB)], xb, sem)
        pltpu.async_copy(ids_hbm.at[pl.ds(base, RB)], ib.at[pl.ds(0, RB)], sem)

    def wait_dma(kb, xb, ib, sem):
        base = kb * RB
        pltpu.make_async_copy(x_hbm.at[pl.ds(base, RB)], xb, sem).wait()
        pltpu.make_async_copy(ids_hbm.at[pl.ds(base, RB)],
                              ib.at[pl.ds(0, RB)], sem).wait()

    def process(kb, xb, ib):
        base = kb * RB
        lo_r = jnp.maximum(rs - base, 0)
        hi_r = jnp.minimum(re - base, RB)

        def chunk_body(c, _):
            cb = c * CH
            full = jnp.logical_and(cb >= lo_r, cb + CH <= hi_r)

            @pl.when(full)
            def _():
                chunk_work(xb, ib, cb, lo_r, hi_r, False)

            @pl.when(jnp.logical_not(full))
            def _():
                chunk_work(xb, ib, cb, lo_r, hi_r, True)
            return 0

        lax.fori_loop(lo_r // CH, (hi_r + CH - 1) // CH, chunk_body, 0)

    kb_lo = rs // RB
    kb_hi = (re + RB - 1) // RB

    @pl.when(kb_lo < kb_hi)
    def _prologue():
        start_dma(kb_lo, xbuf0, idbuf0, sem0)

    def pair_body(p, _):
        b0 = kb_lo + 2 * p
        b1 = b0 + 1
        wait_dma(b0, xbuf0, idbuf0, sem0)

        @pl.when(b1 < kb_hi)
        def _():
            start_dma(b1, xbuf1, idbuf1, sem1)
        process(b0, xbuf0, idbuf0)

        @pl.when(b1 < kb_hi)
        def _():
            wait_dma(b1, xbuf1, idbuf1, sem1)

            @pl.when(b1 + 1 < kb_hi)
            def _():
                start_dma(b1 + 1, xbuf0, idbuf0, sem0)
            process(b1, xbuf1, idbuf1)
        return 0

    npairs = (kb_hi - kb_lo + 1) // 2
    lax.fori_loop(0, npairs, pair_body, 0)

    def fin_body(cidx, _):
        cload = cntbuf[pl.ds(cidx * 8, 16)]
        for s in range(8):
            row = cidx * 8 + s
            cs = cload[s]
            denom = jnp.maximum(jnp.full((16,), cs, jnp.int32),
                                1).astype(jnp.float32)
            for j in range(NV):
                sl = pl.ds(16 * j, 16)
                sv = sumacc[row, sl]
                stage[s, sl] = sv
                stage[s, pl.ds(D + 16 * j, 16)] = sv / denom
                stage[s, pl.ds(2 * D + 16 * j, 16)] = maxacc[row, sl]

            @pl.when(cs == 0)
            def _zero_row():
                for j in range(3 * NV):
                    stage[s, pl.ds(16 * j, 16)] = zero16
        pltpu.sync_copy(stage, out_hbm.at[pl.ds(seg_lo + cidx * 8, 8)])
        return 0
    lax.fori_loop(0, nchunks, fin_body, 0)


@jax.jit
def _run(x, batch_i32, bounds):
    mesh = plsc.VectorSubcoreMesh(core_axis_name="c", subcore_axis_name="s")
    f = pl.kernel(
        _sc_body,
        out_type=jax.ShapeDtypeStruct((B, 3 * D), jnp.float32),
        mesh=mesh,
        scratch_types=[
            pltpu.VMEM((16,), jnp.int32),
            pltpu.VMEM((RB, D), jnp.float32),
            pltpu.VMEM((RB, D), jnp.float32),
            pltpu.VMEM((RB + 16,), jnp.int32),
            pltpu.VMEM((RB + 16,), jnp.int32),
            pltpu.VMEM((SEG_MAX, D), jnp.float32),
            pltpu.VMEM((SEG_MAX, D), jnp.float32),
            pltpu.VMEM((SEG_MAX + 16,), jnp.int32),
            pltpu.VMEM((8, 3 * D), jnp.float32),
            pltpu.SemaphoreType.DMA,
            pltpu.SemaphoreType.DMA,
        ],
    )
    return f(x, batch_i32, bounds)


def kernel(x, batch, batch_size):
    ids = batch.astype(jnp.int32)
    bvals = jnp.array([SEG_BASE * w for w in range(NW)] + [B], jnp.int32)
    bnds = jnp.searchsorted(ids, bvals, side="left").astype(jnp.int32)
    bounds = jnp.zeros((NW, 16), jnp.int32)
    bounds = bounds.at[:, 0].set(bnds[:NW]).at[:, 1].set(bnds[1:])
    return _run(x, ids, bounds)
